# Initial kernel scaffold; baseline (speedup 1.0000x reference)
import jax
import jax.numpy as jnp
from jax.experimental import pallas as pl


def kernel(x, adj, W1, b1, W2, b2, W3, b3):
    raise NotImplementedError("write your pallas kernel here")


# speedup vs baseline: 1.0237x; 1.0237x over previous
import jax
import jax.numpy as jnp
from jax.experimental import pallas as pl
from functools import partial

_BR1 = 400
_BR2 = 400


def _xw_kernel(h_ref, w_ref, out_ref):
    out_ref[...] = jnp.dot(
        h_ref[...], w_ref[...], preferred_element_type=jnp.float32
    ).astype(jnp.bfloat16)


def _xw(h, w):
    n, _ = h.shape
    o = w.shape[1]
    return pl.pallas_call(
        _xw_kernel,
        out_shape=jax.ShapeDtypeStruct((n, o), jnp.bfloat16),
    )(h, w)


def _l1_kernel(adj_ref, u_ref, b_ref, h_ref, adjb_ref):
    a = adj_ref[...].astype(jnp.bfloat16)
    adjb_ref[...] = a
    acc = jnp.dot(a, u_ref[...], preferred_element_type=jnp.float32)
    acc = acc + b_ref[0:1, :]
    h_ref[...] = jnp.where(acc >= 0, acc, 0.01 * acc)


def _layer1(adj, u, b8):
    n = adj.shape[0]
    f = u.shape[1]
    return pl.pallas_call(
        _l1_kernel,
        grid=(n // _BR1,),
        in_specs=[
            pl.BlockSpec((_BR1, n), lambda i: (i, 0)),
            pl.BlockSpec((n, f), lambda i: (0, 0)),
            pl.BlockSpec((8, f), lambda i: (0, 0)),
        ],
        out_specs=[
            pl.BlockSpec((_BR1, f), lambda i: (i, 0)),
            pl.BlockSpec((_BR1, n), lambda i: (i, 0)),
        ],
        out_shape=[
            jax.ShapeDtypeStruct((n, f), jnp.float32),
            jax.ShapeDtypeStruct((n, n), jnp.bfloat16),
        ],
    )(adj, u, b8)


def _lk_kernel(adj_ref, u_ref, b_ref, h_ref, *, act):
    acc = jnp.dot(adj_ref[...], u_ref[...], preferred_element_type=jnp.float32)
    acc = acc + b_ref[0:1, :]
    if act:
        acc = jnp.where(acc >= 0, acc, 0.01 * acc)
    h_ref[...] = acc


def _layerk(adjb, u, b8, act):
    n = adjb.shape[0]
    f = u.shape[1]
    return pl.pallas_call(
        partial(_lk_kernel, act=act),
        grid=(n // _BR2,),
        in_specs=[
            pl.BlockSpec((_BR2, n), lambda i: (i, 0)),
            pl.BlockSpec((n, f), lambda i: (0, 0)),
            pl.BlockSpec((8, f), lambda i: (0, 0)),
        ],
        out_specs=pl.BlockSpec((_BR2, f), lambda i: (i, 0)),
        out_shape=jax.ShapeDtypeStruct((n, f), jnp.float32),
    )(adjb, u, b8)


def kernel(x, adj, W1, b1, W2, b2, W3, b3):
    b1_8 = jnp.tile(b1.reshape(1, -1).astype(jnp.float32), (8, 1))
    b2_8 = jnp.tile(b2.reshape(1, -1).astype(jnp.float32), (8, 1))
    b3_8 = jnp.tile(b3.reshape(1, -1).astype(jnp.float32), (8, 1))

    u1 = _xw(x, W1)
    h1, adjb = _layer1(adj, u1, b1_8)
    u2 = _xw(h1, W2)
    h2 = _layerk(adjb, u2, b2_8, act=True)
    u3 = _xw(h2, W3)
    out = _layerk(adjb, u3, b3_8, act=False)
    return out


# speedup vs baseline: 1.0634x; 1.0388x over previous
import jax
import jax.numpy as jnp
from jax.experimental import pallas as pl
from jax.experimental.pallas import tpu as pltpu
from functools import partial

_BR1 = 400
_BR2 = 400


def _l1_kernel(adj_ref, x_ref, w_ref, b_ref, h_ref, adjb_ref, u_ref):
    @pl.when(pl.program_id(0) == 0)
    def _():
        u_ref[...] = jnp.dot(
            x_ref[...], w_ref[...], preferred_element_type=jnp.float32
        ).astype(jnp.bfloat16)

    a = adj_ref[...].astype(jnp.bfloat16)
    adjb_ref[...] = a
    acc = jnp.dot(a, u_ref[...], preferred_element_type=jnp.float32)
    acc = acc + b_ref[...]
    h_ref[...] = jnp.where(acc >= 0, acc, 0.01 * acc)


def _layer1(adj, x, w, b):
    n = adj.shape[0]
    f = w.shape[1]
    fin = x.shape[1]
    return pl.pallas_call(
        _l1_kernel,
        grid=(n // _BR1,),
        in_specs=[
            pl.BlockSpec((_BR1, n), lambda i: (i, 0)),
            pl.BlockSpec((n, fin), lambda i: (0, 0)),
            pl.BlockSpec((fin, f), lambda i: (0, 0)),
            pl.BlockSpec((1, f), lambda i: (0, 0)),
        ],
        out_specs=[
            pl.BlockSpec((_BR1, f), lambda i: (i, 0)),
            pl.BlockSpec((_BR1, n), lambda i: (i, 0)),
        ],
        out_shape=[
            jax.ShapeDtypeStruct((n, f), jnp.float32),
            jax.ShapeDtypeStruct((n, n), jnp.bfloat16),
        ],
        scratch_shapes=[pltpu.VMEM((n, f), jnp.bfloat16)],
    )(adj, x, w, b)


def _lk_kernel(adjb_ref, h_ref, w_ref, b_ref, o_ref, u_ref, *, act):
    @pl.when(pl.program_id(0) == 0)
    def _():
        u_ref[...] = jnp.dot(
            h_ref[...], w_ref[...], preferred_element_type=jnp.float32
        ).astype(jnp.bfloat16)

    acc = jnp.dot(adjb_ref[...], u_ref[...], preferred_element_type=jnp.float32)
    acc = acc + b_ref[...]
    if act:
        acc = jnp.where(acc >= 0, acc, 0.01 * acc)
    o_ref[...] = acc


def _layerk(adjb, h, w, b, act):
    n = adjb.shape[0]
    f = w.shape[1]
    fin = h.shape[1]
    return pl.pallas_call(
        partial(_lk_kernel, act=act),
        grid=(n // _BR2,),
        in_specs=[
            pl.BlockSpec((_BR2, n), lambda i: (i, 0)),
            pl.BlockSpec((n, fin), lambda i: (0, 0)),
            pl.BlockSpec((fin, f), lambda i: (0, 0)),
            pl.BlockSpec((1, f), lambda i: (0, 0)),
        ],
        out_specs=pl.BlockSpec((_BR2, f), lambda i: (i, 0)),
        out_shape=jax.ShapeDtypeStruct((n, f), jnp.float32),
        scratch_shapes=[pltpu.VMEM((n, f), jnp.bfloat16)],
    )(adjb, h, w, b)


def kernel(x, adj, W1, b1, W2, b2, W3, b3):
    h1, adjb = _layer1(adj, x, W1, b1.reshape(1, -1))
    h2 = _layerk(adjb, h1, W2, b2.reshape(1, -1), act=True)
    out = _layerk(adjb, h2, W3, b3.reshape(1, -1), act=False)
    return out
